# Initial kernel scaffold; baseline (speedup 1.0000x reference)
import jax
import jax.numpy as jnp
from jax.experimental import pallas as pl


def kernel(x, c_0, c_1, c_2):
    raise NotImplementedError("write your pallas kernel here")


# speedup vs baseline: 5.5753x; 5.5753x over previous
import functools

import jax
import jax.numpy as jnp
from jax import lax
from jax.experimental import pallas as pl
from jax.experimental.pallas import tpu as pltpu
from jax.experimental.pallas import tpu_sc as plsc

N_ROWS = 16384
N_COLS = 16384
K = 163
KP = 176
KH = KP // 2
BATCH = 256
NC = 2
NS = 16
NW = NC * NS
ROWS_PER_W = N_ROWS // NW
LANES = 16
VB = BATCH // LANES


def _spmv_sc(xT, vals, cols):
    mesh = plsc.VectorSubcoreMesh(core_axis_name="c", subcore_axis_name="s")

    @functools.partial(
        pl.kernel,
        mesh=mesh,
        out_type=jax.ShapeDtypeStruct((N_ROWS, BATCH), jnp.float32),
        scratch_types=[
            pltpu.VMEM((2, KH), jnp.int32),
            pltpu.VMEM((KP,), jnp.float32),
            pltpu.VMEM((KP, BATCH), jnp.float32),
            pltpu.VMEM((BATCH,), jnp.float32),
            pltpu.SemaphoreType.DMA,
        ],
    )
    def k(xT_hbm, vals_hbm, cols_hbm, out_hbm, idx_v, vals_v, rows_v,
          out_stage, sem):
        wid = lax.axis_index("s") * NC + lax.axis_index("c")
        row0 = wid * ROWS_PER_W

        def row_body(i, carry):
            row = row0 + i
            pltpu.sync_copy(cols_hbm.at[row], idx_v)
            pltpu.sync_copy(vals_hbm.at[row], vals_v)
            cp1 = pltpu.async_copy(xT_hbm.at[idx_v.at[0]],
                                   rows_v.at[pl.ds(0, KH)], sem)
            cp2 = pltpu.async_copy(xT_hbm.at[idx_v.at[1]],
                                   rows_v.at[pl.ds(KH, KH)], sem)
            cp1.wait()
            cp2.wait()

            def j_body(jb, accs):
                j0 = pl.multiple_of(jb * LANES, LANES)
                vblock = vals_v[pl.ds(j0, LANES)]
                for t in range(LANES):
                    vv = jnp.full((LANES,), vblock[t], jnp.float32)
                    accs = tuple(
                        accs[k] + vv * rows_v[j0 + t, pl.ds(k * LANES, LANES)]
                        for k in range(VB))
                return accs

            accs = lax.fori_loop(
                0, KP // LANES, j_body,
                tuple(jnp.zeros((LANES,), jnp.float32) for _ in range(VB)))
            for t in range(VB):
                out_stage[pl.ds(t * LANES, LANES)] = accs[t]
            pltpu.sync_copy(out_stage, out_hbm.at[row])
            return carry

        lax.fori_loop(0, ROWS_PER_W, row_body, 0)

    return k(xT, vals, cols)


@jax.jit
def kernel(x, c_0, c_1, c_2):
    del c_2
    xT = x.T
    vals = jnp.zeros((N_ROWS, KP), jnp.float32)
    vals = vals.at[:, :K].set(c_0.reshape(N_ROWS, K))
    cols = jnp.zeros((N_ROWS, KP), jnp.int32)
    cols = cols.at[:, :K].set(c_1.reshape(N_ROWS, K))
    cols = cols.reshape(N_ROWS, 2, KH)
    yT = _spmv_sc(xT, vals, cols)
    return yT.T
